# Initial kernel scaffold; baseline (speedup 1.0000x reference)
import jax
import jax.numpy as jnp
from jax.experimental import pallas as pl


def kernel(x, e1_w, e1_b, e2_w, e2_b, e3_w, e3_b, e4_w, e4_b, d1_w, d1_b, d2_w, d2_b, d3_w, d3_b, d4_w, d4_b):
    raise NotImplementedError("write your pallas kernel here")


# speedup vs baseline: 13.6341x; 13.6341x over previous
import functools

import jax
import jax.numpy as jnp
from jax.experimental import pallas as pl
from jax.experimental.pallas import tpu as pltpu

_TAPS = ((0, 0), (0, 1), (1, 0), (1, 1))


def _enc_body(x_ref, w_ref, b_ref, o_ref, *, Ho, Wo):
    xv = x_ref[0]
    acc = None
    for t, (a, b) in enumerate(_TAPS):
        p = xv[a:a + Ho, b:b + Wo, :].reshape(Ho * Wo, xv.shape[-1])
        d = jnp.dot(p, w_ref[t], preferred_element_type=jnp.float32)
        acc = d if acc is None else acc + d
    y = jnp.maximum(acc + b_ref[...].astype(jnp.float32), 0.0)
    o_ref[0] = y.reshape(Ho, Wo, -1).astype(o_ref.dtype)


def _dec_body(x_ref, w_ref, b_ref, o_ref, *, H, W, act):
    xv = x_ref[0]
    bias = b_ref[...].astype(jnp.float32)
    for g, (ph, pw) in enumerate(_TAPS):
        acc = None
        for t, (a, b) in enumerate(_TAPS):
            p = xv[ph + a:ph + a + H, pw + b:pw + b + W, :]
            p = p.reshape(H * W, xv.shape[-1])
            d = jnp.dot(p, w_ref[g * 4 + t], preferred_element_type=jnp.float32)
            acc = d if acc is None else acc + d
        y = act(acc + bias)
        o_ref[0, g] = y.reshape(H, W, -1).astype(o_ref.dtype)


def _dec_packed_body(x_ref, w_ref, b_ref, o_ref, *, H, W, act):
    C = x_ref.shape[-1]
    chunk = H
    while chunk * W * 128 * 4 > 4 * 1024 * 1024:
        chunk //= 2
    bias = b_ref[...].astype(jnp.float32)
    for h0 in range(0, H, chunk):
        acc = None
        for r in range(3):
            for s in range(3):
                p = x_ref[0, r + h0:r + h0 + chunk, s:s + W, :]
                p = p.reshape(chunk * W, C)
                d = jnp.dot(p, w_ref[r * 3 + s], preferred_element_type=jnp.float32)
                acc = d if acc is None else acc + d
        y = act(acc + bias)
        o_ref[0, h0:h0 + chunk] = y.reshape(chunk, W, -1).astype(o_ref.dtype)


def _dec_skip_body(x_ref, w_ref, b_ref, s_ref, o_ref, *, H, W, act):
    xv = x_ref[0]
    bias = b_ref[...].astype(jnp.float32)
    for g, (ph, pw) in enumerate(_TAPS):
        acc = None
        for t, (a, b) in enumerate(_TAPS):
            p = xv[ph + a:ph + a + H, pw + b:pw + b + W, :]
            p = p.reshape(H * W, xv.shape[-1])
            d = jnp.dot(p, w_ref[g * 4 + t], preferred_element_type=jnp.float32)
            acc = d if acc is None else acc + d
        sk = s_ref[0, g].astype(jnp.float32).reshape(H * W, -1)
        y = act(acc + bias + sk)
        o_ref[0, g] = y.reshape(H, W, -1).astype(o_ref.dtype)


def _conv_enc(x, w, b):
    N, H, W, C = x.shape
    Ho, Wo = H // 2, W // 2
    Co = w.shape[0]
    xp = jnp.pad(x, ((0, 0), (1, 1), (1, 1), (0, 0)))
    s = xp.reshape(N, Ho + 1, 2, Wo + 1, 2, C)
    s = s.transpose(0, 1, 3, 2, 4, 5).reshape(N, Ho + 1, Wo + 1, 4 * C)
    wt = w.transpose(2, 3, 1, 0).reshape(2, 2, 2, 2, C, Co)
    wt = wt.transpose(0, 2, 1, 3, 4, 5).reshape(4, 4 * C, Co).astype(jnp.bfloat16)
    out = pl.pallas_call(
        functools.partial(_enc_body, Ho=Ho, Wo=Wo),
        out_shape=jax.ShapeDtypeStruct((N, Ho, Wo, Co), jnp.bfloat16),
        grid=(N,),
        in_specs=[
            pl.BlockSpec((1, Ho + 1, Wo + 1, 4 * C), lambda n: (n, 0, 0, 0)),
            pl.BlockSpec((4, 4 * C, Co), lambda n: (0, 0, 0)),
            pl.BlockSpec((1, Co), lambda n: (0, 0)),
        ],
        out_specs=pl.BlockSpec((1, Ho, Wo, Co), lambda n: (n, 0, 0, 0)),
        compiler_params=pltpu.CompilerParams(
            dimension_semantics=("parallel",),
        ),
    )(s, wt, b.reshape(1, Co).astype(jnp.float32))
    return out


def _conv_dec(x, w, b, act, skip=None, out_dtype=jnp.bfloat16):
    N, H, W, C = x.shape
    Co = w.shape[1]
    xp = jnp.pad(x, ((0, 0), (1, 1), (1, 1), (0, 0)))
    w2 = w[:, :, ::-1, ::-1]
    wt = w2.transpose(2, 3, 0, 1).reshape(2, 2, 2, 2, C, Co)
    wt = wt.transpose(1, 3, 0, 2, 4, 5).reshape(16, C, Co).astype(jnp.bfloat16)
    bias = b.reshape(1, Co).astype(jnp.float32)

    if skip is None and 4 * Co <= 128:
        wp = jnp.zeros((3, 3, C, 4, Co), wt.dtype)
        for g, (ph, pw) in enumerate(_TAPS):
            for t, (a, b) in enumerate(_TAPS):
                wp = wp.at[ph + a, pw + b, :, g, :].set(wt[g * 4 + t])
        wp = wp.reshape(9, C, 4 * Co)
        o = pl.pallas_call(
            functools.partial(_dec_packed_body, H=H, W=W, act=act),
            out_shape=jax.ShapeDtypeStruct((N, H, W, 4 * Co), out_dtype),
            grid=(N,),
            in_specs=[
                pl.BlockSpec((1, H + 2, W + 2, C), lambda n: (n, 0, 0, 0)),
                pl.BlockSpec((9, C, 4 * Co), lambda n: (0, 0, 0)),
                pl.BlockSpec((1, 4 * Co), lambda n: (0, 0)),
            ],
            out_specs=pl.BlockSpec((1, H, W, 4 * Co), lambda n: (n, 0, 0, 0)),
            compiler_params=pltpu.CompilerParams(
                dimension_semantics=("parallel",),
            ),
        )(xp, wp, jnp.tile(bias, (1, 4)))
        o = o.reshape(N, H, W, 2, 2, Co).transpose(0, 1, 3, 2, 4, 5)
        return o.reshape(N, 2 * H, 2 * W, Co)

    in_specs = [
        pl.BlockSpec((1, H + 2, W + 2, C), lambda n: (n, 0, 0, 0)),
        pl.BlockSpec((16, C, Co), lambda n: (0, 0, 0)),
        pl.BlockSpec((1, Co), lambda n: (0, 0)),
    ]
    args = [xp, wt, bias]
    if skip is None:
        body = functools.partial(_dec_body, H=H, W=W, act=act)
    else:
        sk = skip.reshape(N, H, 2, W, 2, Co)
        sk = sk.transpose(0, 2, 4, 1, 3, 5).reshape(N, 4, H, W, Co)
        in_specs.append(pl.BlockSpec((1, 4, H, W, Co), lambda n: (n, 0, 0, 0, 0)))
        args.append(sk)
        body = functools.partial(_dec_skip_body, H=H, W=W, act=act)

    o = pl.pallas_call(
        body,
        out_shape=jax.ShapeDtypeStruct((N, 4, H, W, Co), out_dtype),
        grid=(N,),
        in_specs=in_specs,
        out_specs=pl.BlockSpec((1, 4, H, W, Co), lambda n: (n, 0, 0, 0, 0)),
        compiler_params=pltpu.CompilerParams(
            dimension_semantics=("parallel",),
        ),
    )(*args)
    o = o.reshape(N, 2, 2, H, W, Co).transpose(0, 3, 1, 4, 2, 5)
    return o.reshape(N, 2 * H, 2 * W, Co)


def kernel(x, e1_w, e1_b, e2_w, e2_b, e3_w, e3_b, e4_w, e4_b,
           d1_w, d1_b, d2_w, d2_b, d3_w, d3_b, d4_w, d4_b):
    relu = functools.partial(jnp.maximum, 0.0)
    xh = x.transpose(0, 2, 3, 1).astype(jnp.bfloat16)
    x1 = _conv_enc(xh, e1_w, e1_b)
    x2 = _conv_enc(x1, e2_w, e2_b)
    x3 = _conv_enc(x2, e3_w, e3_b)
    x4 = _conv_enc(x3, e4_w, e4_b)
    d1 = _conv_dec(x4, d1_w, d1_b, relu, skip=x3)
    d2 = _conv_dec(d1, d2_w, d2_b, relu)
    d3 = _conv_dec(d2, d3_w, d3_b, relu)
    out = _conv_dec(d3, d4_w, d4_b, jax.nn.sigmoid, out_dtype=jnp.float32)
    return out.transpose(0, 3, 1, 2)
